# baseline (speedup 1.0000x reference)
import functools
from math import sqrt

import jax
import jax.numpy as jnp
import numpy as np
from jax import lax
from jax.experimental import pallas as pl
from jax.experimental.pallas import tpu as pltpu
from jax.experimental.pallas import tpu_sc as plsc

D_MODEL = 1024
N_HEADS = 16
DH = D_MODEL // N_HEADS
L_SEQ = 2048
U_TOP = 40

_IDX_SAMPLE = np.asarray(
    jax.random.randint(jax.random.key(1234), (L_SEQ, U_TOP), 0, L_SEQ)
)
_CNT = np.zeros((L_SEQ, L_SEQ), np.int8)
np.add.at(_CNT, (np.arange(L_SEQ)[:, None], _IDX_SAMPLE), 1)
_CNT.setflags(write=False)

_NEG_INF = float("-inf")


def _proj_body(x_ref, wq_ref, wkv_ref, b_ref, qkv_ref, vsum_ref):
    c = pl.program_id(0)
    row = pl.program_id(1)

    @pl.when(c == 0)
    def _():
        qkv_ref[...] = jax.lax.dot_general(
            x_ref[...], wq_ref[...], (((1,), (1,)), ((), ())),
            preferred_element_type=jnp.float32,
        ) + b_ref[0]

    @pl.when(c != 0)
    def _():
        qkv_ref[...] = jax.lax.dot_general(
            x_ref[...], wkv_ref[...], (((1,), (1,)), ((), ())),
            preferred_element_type=jnp.float32,
        ) + b_ref[0]

    cs = jnp.sum(qkv_ref[...], axis=0, keepdims=True)[None]

    @pl.when(row == 0)
    def _():
        vsum_ref[...] = cs

    @pl.when(row != 0)
    def _():
        vsum_ref[...] += cs


def _projection(x2d, wq, wkv, b_cat3):
    return pl.pallas_call(
        _proj_body,
        grid=(3, 8),
        in_specs=[
            pl.BlockSpec((256, 1024), lambda c, r: (r, 0)),
            pl.BlockSpec((1024, 1024), lambda c, r: (0, 0)),
            pl.BlockSpec(
                (1024, 1024),
                lambda c, r: (jnp.maximum(c - 1, 0), 0),
            ),
            pl.BlockSpec((1, 1, 1024), lambda c, r: (c, 0, 0)),
        ],
        out_specs=[
            pl.BlockSpec((256, 1024), lambda c, r: (r, c)),
            pl.BlockSpec((1, 1, 1024), lambda c, r: (c, 0, 0)),
        ],
        out_shape=[
            jax.ShapeDtypeStruct((L_SEQ, 3 * D_MODEL), jnp.float32),
            jax.ShapeDtypeStruct((3, 1, D_MODEL), jnp.float32),
        ],
    )(x2d, wq, wkv, b_cat3)


def _m_body(q_ref, k_ref, cnt_ref, m_ref):
    c = cnt_ref[...].astype(jnp.float32)
    sampled = c > 0.0
    for hh in range(8):
        q = q_ref[:, hh * DH:(hh + 1) * DH]
        k = k_ref[:, hh * DH:(hh + 1) * DH]
        s = jax.lax.dot_general(
            q, k, (((1,), (1,)), ((), ())),
            preferred_element_type=jnp.float32,
        )
        mx = jnp.max(jnp.where(sampled, s, _NEG_INF), axis=1)
        mean = jnp.sum(s * c, axis=1) * (1.0 / U_TOP)
        m_ref[hh, :] = mx - mean


def _m_scores(qkv, cnt):
    return pl.pallas_call(
        _m_body,
        grid=(2, 8),
        in_specs=[
            pl.BlockSpec((256, 8 * DH), lambda g, r: (r, g)),
            pl.BlockSpec((L_SEQ, 8 * DH), lambda g, r: (0, 2 + g)),
            pl.BlockSpec((256, L_SEQ), lambda g, r: (r, 0)),
        ],
        out_specs=pl.BlockSpec((8, 256), lambda g, r: (g, r)),
        out_shape=jax.ShapeDtypeStruct((N_HEADS, L_SEQ), jnp.float32),
    )(qkv, qkv, cnt)


def _topk_body(m_ref, idx_ref):
    v = m_ref[...]
    iota = jax.lax.broadcasted_iota(jnp.int32, (N_HEADS, L_SEQ), 1)
    for j in range(U_TOP):
        rm = jnp.max(v, axis=1, keepdims=True)
        ci = jnp.min(
            jnp.where(v == rm, iota, L_SEQ), axis=1, keepdims=True
        )
        idx_ref[:, j:j + 1] = ci
        v = jnp.where(iota == ci, _NEG_INF, v)


def _topk(m):
    return pl.pallas_call(
        _topk_body,
        grid=(1,),
        in_specs=[pl.BlockSpec((N_HEADS, L_SEQ), lambda i: (0, 0))],
        out_specs=pl.BlockSpec((N_HEADS, U_TOP), lambda i: (0, 0)),
        out_shape=jax.ShapeDtypeStruct((N_HEADS, U_TOP), jnp.int32),
    )(m)


_GDN = lax.GatherDimensionNumbers(
    offset_dims=(), collapsed_slice_dims=(0,), start_index_map=(0,)
)


def _rot(x, sh):
    perm = lax.rem(lax.iota(jnp.int32, 16) + sh, jnp.int32(16))
    return lax.gather(
        x, perm[:, None], dimension_numbers=_GDN, slice_sizes=(1,),
        mode=lax.GatherScatterMode.PROMISE_IN_BOUNDS,
    )


def _bmax(x):
    for sh in (8, 4, 2, 1):
        x = jnp.maximum(x, _rot(x, sh))
    return x


def _bmin(x):
    for sh in (8, 4, 2, 1):
        x = jnp.minimum(x, _rot(x, sh))
    return x


def _topk_sc_body(m_hbm, out_hbm, row_v, idx_v):
    cid = lax.axis_index("c")
    sid = lax.axis_index("s")

    @pl.when(cid == 0)
    def _():
        pltpu.sync_copy(m_hbm.at[sid], row_v)
        iota = lax.iota(jnp.int32, 16)
        big = jnp.full((16,), 4096, jnp.int32)
        neg = jnp.float32(_NEG_INF)
        cm = []
        for g in range(8):
            acc = jnp.full((16,), neg, jnp.float32)
            for k in range(16):
                ch = row_v[pl.ds((g * 16 + k) * 16, 16)]
                acc = jnp.where(iota == k, _bmax(ch), acc)
            cm.append(acc)
        idxvecs = [jnp.zeros((16,), jnp.int32) for _ in range(3)]
        for i in range(U_TOP):
            gmax = cm[0]
            for g in range(1, 8):
                gmax = jnp.maximum(gmax, cm[g])
            mval = _bmax(gmax)
            hit = big
            for g in range(8):
                hit = jnp.minimum(
                    hit, jnp.where(cm[g] == mval, iota + g * 16, big)
                )
            cstar_v = _bmin(hit)
            cstar = cstar_v[0]
            x = row_v[pl.ds(cstar * 16, 16)]
            lane_v = _bmin(jnp.where(x == mval, iota, jnp.int32(16)))
            sel = cstar_v * 16 + lane_v
            idxvecs[i // 16] = jnp.where(
                iota == (i % 16), sel, idxvecs[i // 16]
            )
            x = jnp.where(iota == lane_v, neg, x)
            row_v[pl.ds(cstar * 16, 16)] = x
            newmax = _bmax(x)
            for g in range(8):
                cm[g] = jnp.where(iota + g * 16 == cstar_v, newmax, cm[g])
        for k in range(3):
            idx_v[pl.ds(k * 16, 16)] = idxvecs[k]
        pltpu.sync_copy(idx_v, out_hbm.at[sid])


def _topk_sc(m):
    f = functools.partial(
        pl.kernel,
        out_type=jax.ShapeDtypeStruct((N_HEADS, 48), jnp.int32),
        mesh=plsc.VectorSubcoreMesh(core_axis_name="c", subcore_axis_name="s"),
        scratch_types=[
            pltpu.VMEM((L_SEQ,), jnp.float32),
            pltpu.VMEM((48,), jnp.int32),
        ],
    )(_topk_sc_body)
    return f(m)[:, :U_TOP]


def _attn_body(idx_ref, q_ref, k_ref, v_ref, vm_ref, wot_ref, bo_ref,
               out_ref, tq_ref, corr_ref, base_ref):
    hp = pl.program_id(0)
    for e in range(2):
        h = 2 * hp + e
        for j in range(U_TOP):
            r = idx_ref[h, j]
            tq_ref[pl.ds(j, 1), :] = q_ref[pl.ds(r, 1), e * DH:(e + 1) * DH]
        scores = jax.lax.dot_general(
            tq_ref[...], k_ref[:, e * DH:(e + 1) * DH],
            (((1,), (1,)), ((), ())),
            preferred_element_type=jnp.float32,
        ) * (1.0 / sqrt(DH))
        mx = jnp.max(scores, axis=1, keepdims=True)
        p = jnp.exp(scores - mx)
        attn = p / jnp.sum(p, axis=1, keepdims=True)
        ctx = jax.lax.dot_general(
            attn, v_ref[:, e * DH:(e + 1) * DH], (((1,), (0,)), ((), ())),
            preferred_element_type=jnp.float32,
        )
        vm = vm_ref[e]
        wslice = wot_ref[pl.ds(h * DH, DH), :]
        corr_ref[pl.ds(h * U_TOP, U_TOP), :] = jax.lax.dot_general(
            ctx - vm, wslice, (((1,), (0,)), ((), ())),
            preferred_element_type=jnp.float32,
        )
        bh = jax.lax.dot_general(
            vm, wslice, (((1,), (0,)), ((), ())),
            preferred_element_type=jnp.float32,
        )

        @pl.when(h == 0)
        def _():
            base_ref[...] = bh + bo_ref[...]

        @pl.when(h != 0)
        def _():
            base_ref[...] += bh

    @pl.when(hp == N_HEADS // 2 - 1)
    def _():
        out_ref[...] = jnp.broadcast_to(base_ref[...], (L_SEQ, D_MODEL))

        def body(j, carry):
            r = idx_ref[j // U_TOP, j % U_TOP]
            out_ref[pl.ds(r, 1), :] += corr_ref[pl.ds(j, 1), :]
            return carry

        jax.lax.fori_loop(0, N_HEADS * U_TOP, body, 0)


def _attn(top_idx, qkv, vmean3, wot, bo2):
    return pl.pallas_call(
        _attn_body,
        grid_spec=pltpu.PrefetchScalarGridSpec(
            num_scalar_prefetch=1,
            grid=(N_HEADS // 2,),
            in_specs=[
                pl.BlockSpec((L_SEQ, 2 * DH), lambda hp, *_: (0, hp)),
                pl.BlockSpec((L_SEQ, 2 * DH), lambda hp, *_: (0, 8 + hp)),
                pl.BlockSpec((L_SEQ, 2 * DH), lambda hp, *_: (0, 16 + hp)),
                pl.BlockSpec((2, 1, DH), lambda hp, *_: (hp, 0, 0)),
                pl.BlockSpec((D_MODEL, D_MODEL), lambda hp, *_: (0, 0)),
                pl.BlockSpec((1, D_MODEL), lambda hp, *_: (0, 0)),
            ],
            out_specs=[
                pl.BlockSpec((L_SEQ, D_MODEL), lambda hp, *_: (0, 0)),
            ],
            scratch_shapes=[
                pltpu.VMEM((U_TOP, DH), jnp.float32),
                pltpu.VMEM((N_HEADS * U_TOP, D_MODEL), jnp.float32),
                pltpu.VMEM((1, D_MODEL), jnp.float32),
            ],
        ),
        out_shape=[
            jax.ShapeDtypeStruct((L_SEQ, D_MODEL), jnp.float32),
        ],
    )(top_idx, qkv, qkv, qkv, vmean3, wot, bo2)


def kernel(x, Wq, bq, Wkv, bkv, Wo, bo, factor):
    del factor
    x2d = x.reshape(L_SEQ, D_MODEL)
    b_cat3 = jnp.concatenate([bq, bkv]).reshape(3, 1, D_MODEL)
    cnt = jnp.asarray(_CNT)

    qkv, vsum = _projection(x2d, Wq, Wkv, b_cat3)
    vmean = vsum[2, 0] * (1.0 / L_SEQ)

    m = _m_scores(qkv, cnt)
    top_idx = _topk_sc(m)

    (out,) = _attn(
        top_idx, qkv, vmean.reshape(N_HEADS, 1, DH),
        Wo.T, bo.reshape(1, D_MODEL),
    )
    return out.reshape(1, L_SEQ, D_MODEL)


# speedup vs baseline: 1.1092x; 1.1092x over previous
import functools
from math import sqrt

import jax
import jax.numpy as jnp
import numpy as np
from jax import lax
from jax.experimental import pallas as pl
from jax.experimental.pallas import tpu as pltpu
from jax.experimental.pallas import tpu_sc as plsc

D_MODEL = 1024
N_HEADS = 16
DH = D_MODEL // N_HEADS
L_SEQ = 2048
U_TOP = 40

_IDX_SAMPLE = np.asarray(
    jax.random.randint(jax.random.key(1234), (L_SEQ, U_TOP), 0, L_SEQ)
)
_CNT = np.zeros((L_SEQ, L_SEQ), np.int8)
np.add.at(_CNT, (np.arange(L_SEQ)[:, None], _IDX_SAMPLE), 1)
_CNT.setflags(write=False)

_NEG_INF = float("-inf")


def _proj_body(x_ref, wq_ref, wkv_ref, b_ref, qkv_ref, vsum_ref):
    row = pl.program_id(0)
    x = x_ref[...]
    yq = jax.lax.dot_general(
        x, wq_ref[...], (((1,), (1,)), ((), ())),
        preferred_element_type=jnp.float32,
    ) + b_ref[0, :, :D_MODEL]
    ykv = jax.lax.dot_general(
        x, wkv_ref[...], (((1,), (1,)), ((), ())),
        preferred_element_type=jnp.float32,
    ) + b_ref[0, :, D_MODEL:]
    qkv_ref[:, :D_MODEL] = yq
    qkv_ref[:, D_MODEL:] = ykv
    cs = jnp.concatenate(
        [jnp.sum(yq, axis=0, keepdims=True),
         jnp.sum(ykv, axis=0, keepdims=True)], axis=1,
    )[None]

    @pl.when(row == 0)
    def _():
        vsum_ref[...] = cs

    @pl.when(row != 0)
    def _():
        vsum_ref[...] += cs


def _projection(x2d, wq, wkv, b_cat3):
    return pl.pallas_call(
        _proj_body,
        grid=(8,),
        in_specs=[
            pl.BlockSpec((256, 1024), lambda r: (r, 0)),
            pl.BlockSpec((1024, 1024), lambda r: (0, 0)),
            pl.BlockSpec((2048, 1024), lambda r: (0, 0)),
            pl.BlockSpec((1, 1, 3 * D_MODEL), lambda r: (0, 0, 0)),
        ],
        out_specs=[
            pl.BlockSpec((256, 3 * D_MODEL), lambda r: (r, 0)),
            pl.BlockSpec((1, 1, 3 * D_MODEL), lambda r: (0, 0, 0)),
        ],
        out_shape=[
            jax.ShapeDtypeStruct((L_SEQ, 3 * D_MODEL), jnp.float32),
            jax.ShapeDtypeStruct((1, 1, 3 * D_MODEL), jnp.float32),
        ],
    )(x2d, wq, wkv, b_cat3)


def _m_body(q_ref, k0_ref, k1_ref, cnt_ref, m_ref):
    c = cnt_ref[...].astype(jnp.float32)
    sampled = c > 0.0
    for hh in range(N_HEADS):
        q = q_ref[:, hh * DH:(hh + 1) * DH]
        k_src = k0_ref if hh < 8 else k1_ref
        kk = hh % 8
        k = k_src[:, kk * DH:(kk + 1) * DH]
        s = jax.lax.dot_general(
            q, k, (((1,), (1,)), ((), ())),
            preferred_element_type=jnp.float32,
        )
        mx = jnp.max(jnp.where(sampled, s, _NEG_INF), axis=1)
        mean = jnp.sum(s * c, axis=1) * (1.0 / U_TOP)
        m_ref[hh, :] = mx - mean


def _m_scores(qkv, cnt):
    return pl.pallas_call(
        _m_body,
        grid=(8,),
        in_specs=[
            pl.BlockSpec((256, D_MODEL), lambda r: (r, 0)),
            pl.BlockSpec((L_SEQ, 8 * DH), lambda r: (0, 2)),
            pl.BlockSpec((L_SEQ, 8 * DH), lambda r: (0, 3)),
            pl.BlockSpec((256, L_SEQ), lambda r: (r, 0)),
        ],
        out_specs=pl.BlockSpec((N_HEADS, 256), lambda r: (0, r)),
        out_shape=jax.ShapeDtypeStruct((N_HEADS, L_SEQ), jnp.float32),
    )(qkv, qkv, qkv, cnt)


def _topk_body(m_ref, idx_ref):
    v = m_ref[...]
    iota = jax.lax.broadcasted_iota(jnp.int32, (N_HEADS, L_SEQ), 1)
    for j in range(U_TOP):
        rm = jnp.max(v, axis=1, keepdims=True)
        ci = jnp.min(
            jnp.where(v == rm, iota, L_SEQ), axis=1, keepdims=True
        )
        idx_ref[:, j:j + 1] = ci
        v = jnp.where(iota == ci, _NEG_INF, v)


def _topk(m):
    return pl.pallas_call(
        _topk_body,
        grid=(1,),
        in_specs=[pl.BlockSpec((N_HEADS, L_SEQ), lambda i: (0, 0))],
        out_specs=pl.BlockSpec((N_HEADS, U_TOP), lambda i: (0, 0)),
        out_shape=jax.ShapeDtypeStruct((N_HEADS, U_TOP), jnp.int32),
    )(m)


_GDN = lax.GatherDimensionNumbers(
    offset_dims=(), collapsed_slice_dims=(0,), start_index_map=(0,)
)


def _rot(x, sh):
    perm = lax.rem(lax.iota(jnp.int32, 16) + sh, jnp.int32(16))
    return lax.gather(
        x, perm[:, None], dimension_numbers=_GDN, slice_sizes=(1,),
        mode=lax.GatherScatterMode.PROMISE_IN_BOUNDS,
    )


def _bmax(x):
    for sh in (8, 4, 2, 1):
        x = jnp.maximum(x, _rot(x, sh))
    return x


def _bmin(x):
    for sh in (8, 4, 2, 1):
        x = jnp.minimum(x, _rot(x, sh))
    return x


def _topk_sc_body(m_hbm, out_hbm, row_v, idx_v):
    cid = lax.axis_index("c")
    sid = lax.axis_index("s")

    @pl.when(cid == 0)
    def _():
        pltpu.sync_copy(m_hbm.at[sid], row_v)
        iota = lax.iota(jnp.int32, 16)
        big = jnp.full((16,), 4096, jnp.int32)
        neg = jnp.float32(_NEG_INF)
        cm = []
        for g in range(8):
            acc = jnp.full((16,), neg, jnp.float32)
            for k in range(16):
                ch = row_v[pl.ds((g * 16 + k) * 16, 16)]
                acc = jnp.where(iota == k, _bmax(ch), acc)
            cm.append(acc)
        idxvecs = [jnp.zeros((16,), jnp.int32) for _ in range(3)]
        for i in range(U_TOP):
            gmax = cm[0]
            for g in range(1, 8):
                gmax = jnp.maximum(gmax, cm[g])
            mval = _bmax(gmax)
            hit = big
            for g in range(8):
                hit = jnp.minimum(
                    hit, jnp.where(cm[g] == mval, iota + g * 16, big)
                )
            cstar_v = _bmin(hit)
            cstar = cstar_v[0]
            x = row_v[pl.ds(cstar * 16, 16)]
            lane_v = _bmin(jnp.where(x == mval, iota, jnp.int32(16)))
            sel = cstar_v * 16 + lane_v
            idxvecs[i // 16] = jnp.where(
                iota == (i % 16), sel, idxvecs[i // 16]
            )
            x = jnp.where(iota == lane_v, neg, x)
            row_v[pl.ds(cstar * 16, 16)] = x
            newmax = _bmax(x)
            for g in range(8):
                cm[g] = jnp.where(iota + g * 16 == cstar_v, newmax, cm[g])
        for k in range(3):
            idx_v[pl.ds(k * 16, 16)] = idxvecs[k]
        pltpu.sync_copy(idx_v, out_hbm.at[sid])


def _topk_sc(m):
    f = functools.partial(
        pl.kernel,
        out_type=jax.ShapeDtypeStruct((N_HEADS, 48), jnp.int32),
        mesh=plsc.VectorSubcoreMesh(core_axis_name="c", subcore_axis_name="s"),
        scratch_types=[
            pltpu.VMEM((L_SEQ,), jnp.float32),
            pltpu.VMEM((48,), jnp.int32),
        ],
    )(_topk_sc_body)
    return f(m)[:, :U_TOP]


def _attn_body(idx_ref, q_ref, k_ref, v_ref, vm_ref, wot_ref, bo_ref,
               corr_ref, base_ref, tq_ref):
    hp = pl.program_id(0)
    for e in range(2):
        h = 2 * hp + e
        for j in range(U_TOP):
            r = idx_ref[h, j]
            tq_ref[pl.ds(j, 1), :] = q_ref[pl.ds(r, 1), e * DH:(e + 1) * DH]
        scores = jax.lax.dot_general(
            tq_ref[...], k_ref[:, e * DH:(e + 1) * DH],
            (((1,), (1,)), ((), ())),
            preferred_element_type=jnp.float32,
        ) * (1.0 / sqrt(DH))
        mx = jnp.max(scores, axis=1, keepdims=True)
        p = jnp.exp(scores - mx)
        attn = p / jnp.sum(p, axis=1, keepdims=True)
        ctx = jax.lax.dot_general(
            attn, v_ref[:, e * DH:(e + 1) * DH], (((1,), (0,)), ((), ())),
            preferred_element_type=jnp.float32,
        )
        vm = vm_ref[e]
        wslice = wot_ref[pl.ds(h * DH, DH), :]
        corr_ref[pl.ds(e * U_TOP, U_TOP), :] = jax.lax.dot_general(
            ctx - vm, wslice, (((1,), (0,)), ((), ())),
            preferred_element_type=jnp.float32,
        )
        bh = jax.lax.dot_general(
            vm, wslice, (((1,), (0,)), ((), ())),
            preferred_element_type=jnp.float32,
        )

        @pl.when(h == 0)
        def _():
            base_ref[...] = bh + bo_ref[...]

        @pl.when(h != 0)
        def _():
            base_ref[...] += bh


def _attn(top_idx, qkv, vmean3, wot, bo2):
    return pl.pallas_call(
        _attn_body,
        grid_spec=pltpu.PrefetchScalarGridSpec(
            num_scalar_prefetch=1,
            grid=(N_HEADS // 2,),
            in_specs=[
                pl.BlockSpec((L_SEQ, 2 * DH), lambda hp, *_: (0, hp)),
                pl.BlockSpec((L_SEQ, 2 * DH), lambda hp, *_: (0, 8 + hp)),
                pl.BlockSpec((L_SEQ, 2 * DH), lambda hp, *_: (0, 16 + hp)),
                pl.BlockSpec((2, 1, DH), lambda hp, *_: (hp, 0, 0)),
                pl.BlockSpec((D_MODEL, D_MODEL), lambda hp, *_: (0, 0)),
                pl.BlockSpec((1, D_MODEL), lambda hp, *_: (0, 0)),
            ],
            out_specs=[
                pl.BlockSpec((2 * U_TOP, D_MODEL), lambda hp, *_: (hp, 0)),
                pl.BlockSpec((1, D_MODEL), lambda hp, *_: (0, 0)),
            ],
            scratch_shapes=[pltpu.VMEM((U_TOP, DH), jnp.float32)],
        ),
        out_shape=[
            jax.ShapeDtypeStruct((N_HEADS * U_TOP, D_MODEL), jnp.float32),
            jax.ShapeDtypeStruct((1, D_MODEL), jnp.float32),
        ],
    )(top_idx, qkv, qkv, qkv, vmean3, wot, bo2)


def _assemble_body(idxf_ref, corr_ref, base_ref, out_ref):
    out_ref[...] = jnp.broadcast_to(base_ref[...], (L_SEQ, D_MODEL))

    def body(j, carry):
        r = idxf_ref[j]
        out_ref[pl.ds(r, 1), :] += corr_ref[pl.ds(j, 1), :]
        return carry

    jax.lax.fori_loop(0, N_HEADS * U_TOP, body, 0)


def _assemble(idx_flat, corr, base):
    return pl.pallas_call(
        _assemble_body,
        grid_spec=pltpu.PrefetchScalarGridSpec(
            num_scalar_prefetch=1,
            grid=(1,),
            in_specs=[
                pl.BlockSpec((N_HEADS * U_TOP, D_MODEL), lambda i, *_: (0, 0)),
                pl.BlockSpec((1, D_MODEL), lambda i, *_: (0, 0)),
            ],
            out_specs=pl.BlockSpec((L_SEQ, D_MODEL), lambda i, *_: (0, 0)),
        ),
        out_shape=jax.ShapeDtypeStruct((L_SEQ, D_MODEL), jnp.float32),
    )(idx_flat, corr, base)


def kernel(x, Wq, bq, Wkv, bkv, Wo, bo, factor):
    del factor
    x2d = x.reshape(L_SEQ, D_MODEL)
    b_cat3 = jnp.concatenate([bq, bkv]).reshape(1, 1, 3 * D_MODEL)
    cnt = jnp.asarray(_CNT)

    qkv, vsum = _projection(x2d, Wq, Wkv, b_cat3)
    vmean = vsum[0, 0, 2 * D_MODEL:] * (1.0 / L_SEQ)

    m = _m_scores(qkv, cnt)
    top_idx = _topk_sc(m)

    corr, base = _attn(
        top_idx, qkv, vmean.reshape(N_HEADS, 1, DH),
        Wo.T, bo.reshape(1, D_MODEL),
    )
    out = _assemble(top_idx.reshape(N_HEADS * U_TOP), corr, base)
    return out.reshape(1, L_SEQ, D_MODEL)
